# baseline (speedup 1.0000x reference)
import functools

import jax
import jax.numpy as jnp
from jax import lax
from jax.experimental import pallas as pl
from jax.experimental.pallas import tpu as pltpu
from jax.experimental.pallas import tpu_sc as plsc

_CHUNK = 128


def _z(_):
    return jnp.int32(0)


def _gather_body(chunks_per_tile, t2_hbm, t3_hbm, idx2_hbm, idx3_hbm,
                 e2_hbm, e3_hbm, idx_v, rows_a, rows_b, sem_a, sem_b):
    info = plsc.get_sparse_core_info()
    nc = info.num_cores
    wid = lax.axis_index("s") * jnp.int32(nc) + lax.axis_index("c")
    tpt = chunks_per_tile * _CHUNK
    base = wid * jnp.int32(tpt)

    bufs = [rows_a, rows_b]
    sems = [sem_a, sem_b]
    work = []
    for g, (tab, out, idx) in enumerate(((t2_hbm, e2_hbm, idx2_hbm),
                                         (t3_hbm, e3_hbm, idx3_hbm))):
        pltpu.sync_copy(idx.at[pl.ds(base, tpt)],
                        idx_v.at[pl.ds(jnp.int32(g * tpt), tpt)])
        for c in range(chunks_per_tile):
            work.append((tab, out, g * chunks_per_tile + c, c))

    def start(k):
        tab, _, slot, _ = work[k]
        return pltpu.async_copy(
            tab.at[idx_v.at[pl.ds(jnp.int32(slot * _CHUNK), _CHUNK)]],
            bufs[k % 2], sems[k % 2])

    copies = [start(0), None]
    for k in range(len(work)):
        if k + 1 < len(work):
            copies[(k + 1) % 2] = start(k + 1)
        copies[k % 2].wait()
        _, out, _, c = work[k]
        pltpu.sync_copy(
            bufs[k % 2],
            out.at[pl.ds(base + jnp.int32(c * _CHUNK), _CHUNK)])


def _fused_body(e2_ref, e3_ref, h_ref, w2_ref, w3_ref, gw_ref, o_ref):
    v = jnp.dot(e2_ref[...].astype(jnp.bfloat16), w2_ref[...],
                preferred_element_type=jnp.float32)
    v = v + jnp.dot(e3_ref[...].astype(jnp.bfloat16), w3_ref[...],
                    preferred_element_type=jnp.float32)
    h = h_ref[...]
    hid = h.shape[-1]
    sh = jnp.mean(h * h, axis=-1, keepdims=True)
    sv = jnp.mean(v * v, axis=-1, keepdims=True)
    num = jnp.sum(h * v * gw_ref[...], axis=-1, keepdims=True)
    gate = (num * lax.rsqrt(sh + 1e-6) * lax.rsqrt(sv + 1e-6)
            / (hid ** 0.5))
    gate = jnp.sqrt(jnp.maximum(jnp.abs(gate), 1e-6)) * jnp.sign(gate)
    gate = jax.nn.sigmoid(gate)
    o_ref[...] = gate * v


def kernel(hidden, input_ids, compress_table, hash_mult, tables_2gram,
           tables_3gram, W_v, gate_w_h, gate_w_v):
    b, t, hid = hidden.shape
    nh, ts, ed = tables_2gram.shape
    bt = b * t
    dg = nh * ed

    clamped = jnp.clip(input_ids.astype(jnp.int64), 0,
                       compress_table.shape[0] - 1)
    ids = jnp.take(compress_table, clamped, axis=0)
    shifted_1 = jnp.pad(ids[:, :-1], ((0, 0), (1, 0)), constant_values=0)
    shifted_2 = jnp.pad(ids[:, :-2], ((0, 0), (2, 0)), constant_values=0)
    hash_2 = jnp.bitwise_xor(ids * hash_mult[0], shifted_1 * hash_mult[1])
    hash_3 = jnp.bitwise_xor(hash_2, shifted_2 * hash_mult[2])
    idx2 = jnp.maximum(hash_2 % ts, 0).astype(jnp.int32).reshape(-1)
    idx3 = jnp.maximum(hash_3 % ts, 0).astype(jnp.int32).reshape(-1)

    t2 = jnp.swapaxes(tables_2gram, 0, 1).reshape(ts, dg)
    t3 = jnp.swapaxes(tables_3gram, 0, 1).reshape(ts, dg)

    info = plsc.get_sparse_core_info()
    n_tiles = info.num_cores * info.num_subcores
    chunks_per_tile = bt // (n_tiles * _CHUNK)
    mesh = plsc.VectorSubcoreMesh(core_axis_name="c", subcore_axis_name="s")

    e2, e3 = pl.kernel(
        functools.partial(_gather_body, chunks_per_tile),
        mesh=mesh,
        out_type=[
            jax.ShapeDtypeStruct((bt, dg), jnp.float32),
            jax.ShapeDtypeStruct((bt, dg), jnp.float32),
        ],
        scratch_types=[
            pltpu.VMEM((2 * bt // n_tiles,), jnp.int32),
            pltpu.VMEM((_CHUNK, dg), jnp.float32),
            pltpu.VMEM((_CHUNK, dg), jnp.float32),
            pltpu.SemaphoreType.DMA,
            pltpu.SemaphoreType.DMA,
        ],
    )(t2, t3, idx2, idx3)

    w_t = W_v.T.astype(jnp.bfloat16)
    gw = (gate_w_h * gate_w_v).reshape(1, hid)
    blk = 1024
    out = pl.pallas_call(
        _fused_body,
        grid=(bt // blk,),
        in_specs=[
            pl.BlockSpec((blk, dg), lambda i: (i, _z(i))),
            pl.BlockSpec((blk, dg), lambda i: (i, _z(i))),
            pl.BlockSpec((blk, hid), lambda i: (i, _z(i))),
            pl.BlockSpec((dg, hid), lambda i: (_z(i), _z(i))),
            pl.BlockSpec((dg, hid), lambda i: (_z(i), _z(i))),
            pl.BlockSpec((1, hid), lambda i: (_z(i), _z(i))),
        ],
        out_specs=pl.BlockSpec((blk, hid), lambda i: (i, _z(i))),
        out_shape=jax.ShapeDtypeStruct((bt, hid), jnp.float32),
    )(e2, e3, hidden.reshape(bt, hid), w_t[:dg], w_t[dg:], gw)

    return out.reshape(b, t, hid)
